# baseline (speedup 1.0000x reference)
import functools

import jax
import jax.numpy as jnp
from jax import lax
from jax.experimental import pallas as pl
from jax.experimental.pallas import tpu as pltpu
from jax.experimental.pallas import tpu_sc as plsc

_N = 10000
_HALF = 5000
_P = 2500
_EMB = 128
_E = 320000
_CHUNK = 120
_NCORE = 2
_NSUB = 16
_NW = _NCORE * _NSUB
_K0 = 168
_RING = 4
_EPROWS = _NSUB * _K0
_EP = _EPROWS * _CHUNK
_TROWS = 10240
_PAD_SRC = 10016
_AROWS = 5120


def _make_scatter_kernel():
    mesh = plsc.VectorSubcoreMesh(core_axis_name="c", subcore_axis_name="s")

    @functools.partial(
        pl.kernel,
        out_type=jax.ShapeDtypeStruct((1, _HALF, _EMB), jnp.float32),
        mesh=mesh,
        scratch_types=[
            pltpu.VMEM((_K0, _CHUNK), jnp.int32),
            pltpu.VMEM((16, _CHUNK), jnp.int32),
            pltpu.VMEM((16,), jnp.int32),
            pltpu.VMEM((_RING, _CHUNK, _EMB), jnp.float32),
            pltpu.VMEM_SHARED((_AROWS, _EMB), jnp.float32),
            pltpu.SemaphoreType.DMA, pltpu.SemaphoreType.DMA,
            pltpu.SemaphoreType.DMA, pltpu.SemaphoreType.DMA,
            pltpu.SemaphoreType.DMA, pltpu.SemaphoreType.DMA,
            pltpu.SemaphoreType.DMA, pltpu.SemaphoreType.DMA,
            pltpu.SemaphoreType.DMA,
        ],
    )
    def k(table_hbm, src_hbm, dst_hbm, nrows_hbm, zeros_hbm, out_hbm,
          src_v, dst_v, nrows_s, bufs, acc, *sems):
        gsem, ssem = sems[:_RING], sems[_RING:2 * _RING]
        dsem = sems[-1]
        core = lax.axis_index("c")
        sid = lax.axis_index("s")

        @pl.when(core == 0)
        def _():
            base = sid * _K0
            zrows = _AROWS // _NSUB
            pltpu.sync_copy(zeros_hbm.at[pl.ds(sid * zrows, zrows)],
                            acc.at[pl.ds(sid * zrows, zrows)])
            pltpu.sync_copy(nrows_hbm, nrows_s)
            nrows = nrows_s[...][0]
            myc = (jnp.maximum(nrows - sid, 0) + 15) // 16
            ng = jnp.maximum((myc + 7) // 8, 2)
            kk = ng * 8
            pltpu.sync_copy(src_hbm.at[pl.ds(base, _K0)], src_v)
            pltpu.sync_copy(dst_hbm.at[pl.ds(base, 8)], dst_v.at[pl.ds(0, 8)])
            pltpu.async_copy(dst_hbm.at[pl.ds(base + 8, 8)],
                             dst_v.at[pl.ds(8, 8)], dsem)
            for b in range(_RING):
                pltpu.async_copy(table_hbm.at[src_v.at[b]], bufs.at[b],
                                 gsem[b])

            def group(g, _):
                ci = g * 8
                p = g % 2

                @pl.when(ci >= 8)
                def _():
                    pltpu.make_async_copy(dst_hbm.at[pl.ds(base, 8)],
                                          dst_v.at[pl.ds(0, 8)], dsem).wait()

                for half in range(2):
                    for b in range(_RING):
                        j = 4 * half + b
                        pltpu.make_async_copy(table_hbm.at[src_v.at[ci + j]],
                                              bufs.at[b], gsem[b]).wait()
                        pltpu.async_copy(bufs.at[b],
                                         acc.at[dst_v.at[p * 8 + j]],
                                         ssem[b], add=True)
                    for b in range(_RING):
                        j = 4 * half + b
                        k = ci + j
                        pltpu.make_async_copy(bufs.at[b],
                                              acc.at[dst_v.at[p * 8 + j]],
                                              ssem[b]).wait()

                        @pl.when(k + _RING < kk)
                        def _():
                            pltpu.async_copy(table_hbm.at[src_v.at[k + _RING]],
                                             bufs.at[b], gsem[b])

                @pl.when(ci + 16 < kk)
                def _():
                    pltpu.async_copy(dst_hbm.at[pl.ds(base + ci + 16, 8)],
                                     dst_v.at[pl.ds(p * 8, 8)], dsem)
                return None

            lax.fori_loop(0, ng, group, None)

        plsc.subcore_barrier()

        @pl.when((core == 0) & (sid < 5))
        def _():
            pltpu.sync_copy(acc.at[pl.ds(sid * 1000, 1000)],
                            out_hbm.at[0].at[pl.ds(sid * 1000, 1000)])

    return k


_scatter = _make_scatter_kernel()


def _full_spec():
    return pl.BlockSpec((_EMB, _EMB), lambda i: (0, 0))


def _bias_spec():
    return pl.BlockSpec((1, _EMB), lambda i: (0, 0))


def _row_spec(rows):
    return pl.BlockSpec((rows, _EMB), lambda i: (i, 0))


def _dot(a, b):
    return jnp.dot(a, b, preferred_element_type=jnp.float32)


def _mlp3(x, params):
    (w1, b1), (w2, b2), (w3, b3) = params
    rows = 1000
    n = x.shape[0]

    def body(x_ref, w1_ref, b1_ref, w2_ref, b2_ref, w3_ref, b3_ref, o_ref):
        h = jnp.maximum(_dot(x_ref[...], w1_ref[...]) + b1_ref[...], 0.0)
        h = jnp.maximum(_dot(h, w2_ref[...]) + b2_ref[...], 0.0)
        o_ref[...] = _dot(h, w3_ref[...]) + b3_ref[...]

    return pl.pallas_call(
        body,
        grid=(n // rows,),
        in_specs=[_row_spec(rows), _full_spec(), _bias_spec(), _full_spec(),
                  _bias_spec(), _full_spec(), _bias_spec()],
        out_specs=_row_spec(rows),
        out_shape=jax.ShapeDtypeStruct((n, _EMB), jnp.float32),
    )(x, w1, b1.reshape(1, _EMB), w2, b2.reshape(1, _EMB),
      w3, b3.reshape(1, _EMB))


def _clause_update(emb_c, p0, cu_params, cm_params):
    (wu1, bu1), (wu2, bu2), (wu3, bu3) = cu_params
    (wm1, bm1), (wm2, bm2), (wm3, bm3) = cm_params
    wu1a, wu1b = wu1[:_EMB], wu1[_EMB:]
    rows = 1000

    def body(e_ref, p0_ref, wu1a_ref, wu1b_ref, bu1_ref, wu2_ref,
             bu2_ref, wu3_ref, bu3_ref, wm1_ref, bm1_ref, wm2_ref, bm2_ref,
             wm3_ref, bm3_ref, ce_ref, cm_ref):
        h = _dot(e_ref[...], wu1a_ref[...]) + _dot(p0_ref[...], wu1b_ref[...])
        h = jnp.maximum(h + bu1_ref[...], 0.0)
        h = jnp.maximum(_dot(h, wu2_ref[...]) + bu2_ref[...], 0.0)
        ce = _dot(h, wu3_ref[...]) + bu3_ref[...]
        ce_ref[...] = ce
        m = jnp.maximum(_dot(ce, wm1_ref[...]) + bm1_ref[...], 0.0)
        m = jnp.maximum(_dot(m, wm2_ref[...]) + bm2_ref[...], 0.0)
        cm_ref[...] = _dot(m, wm3_ref[...]) + bm3_ref[...]

    return pl.pallas_call(
        body,
        grid=(_HALF // rows,),
        in_specs=[_row_spec(rows), _row_spec(rows),
                  _full_spec(), _full_spec(), _bias_spec(),
                  _full_spec(), _bias_spec(), _full_spec(), _bias_spec(),
                  _full_spec(), _bias_spec(), _full_spec(), _bias_spec(),
                  _full_spec(), _bias_spec()],
        out_specs=[_row_spec(rows), _row_spec(rows)],
        out_shape=[jax.ShapeDtypeStruct((_HALF, _EMB), jnp.float32),
                   jax.ShapeDtypeStruct((_HALF, _EMB), jnp.float32)],
    )(emb_c, p0, wu1a, wu1b, bu1.reshape(1, _EMB), wu2,
      bu2.reshape(1, _EMB), wu3, bu3.reshape(1, _EMB), wm1,
      bm1.reshape(1, _EMB), wm2, bm2.reshape(1, _EMB), wm3,
      bm3.reshape(1, _EMB))


def _literal_update(emb_l, q0, l_msg, lu_params):
    (wl1, bl1), (wl2, bl2), (wl3, bl3) = lu_params
    wl1a, wl1b, wl1c = wl1[:_EMB], wl1[_EMB:2 * _EMB], wl1[2 * _EMB:]

    def body(e_ref, q0_ref, f_ref, wl1a_ref, wl1b_ref, wl1c_ref,
             bl1_ref, wl2_ref, bl2_ref, wl3_ref, bl3_ref, o_ref):
        h = (_dot(e_ref[0], wl1a_ref[...]) + _dot(q0_ref[0], wl1b_ref[...])
             + _dot(f_ref[0], wl1c_ref[...]))
        h = jnp.maximum(h + bl1_ref[...], 0.0)
        h = jnp.maximum(_dot(h, wl2_ref[...]) + bl2_ref[...], 0.0)
        o_ref[0] = _dot(h, wl3_ref[...]) + bl3_ref[...]

    h3 = pl.BlockSpec((1, _P, _EMB), lambda j: (j, 0, 0))
    flip_spec = pl.BlockSpec((1, _P, _EMB), lambda j: ((j + 1) % 2, 0, 0))
    r3 = lambda a: a.reshape(2, _P, _EMB)
    out = pl.pallas_call(
        body,
        grid=(2,),
        in_specs=[h3, h3, flip_spec,
                  _full_spec(), _full_spec(), _full_spec(), _bias_spec(),
                  _full_spec(), _bias_spec(), _full_spec(), _bias_spec()],
        out_specs=h3,
        out_shape=jax.ShapeDtypeStruct((2, _P, _EMB), jnp.float32),
    )(r3(emb_l), r3(q0), r3(l_msg), wl1a, wl1b, wl1c,
      bl1.reshape(1, _EMB), wl2, bl2.reshape(1, _EMB), wl3,
      bl3.reshape(1, _EMB))
    return out.reshape(_HALF, _EMB)


def kernel(node_embedding, node_type, edge_index, L_msg, C_msg, L_update,
           C_update):
    del node_type
    emb_l = node_embedding[:_HALF]
    emb_c = node_embedding[_HALF:]
    src, dst = edge_index[0], edge_index[1]
    dump = _HALF + (jnp.arange(_EP, dtype=jnp.int32) % (_AROWS - _HALF))

    def compact(keep, dst_w):
        pos = jnp.cumsum(keep.astype(jnp.int32)) - 1
        wr = jnp.where(keep, pos, _EP)
        srcc = jnp.full((_EP,), _PAD_SRC, jnp.int32).at[wr].set(
            src, mode="drop")
        dstc = dump.at[wr].set(dst_w, mode="drop")
        s3 = srcc.reshape(_K0, _NSUB, _CHUNK).transpose(1, 0, 2)
        d3 = dstc.reshape(_K0, _NSUB, _CHUNK).transpose(1, 0, 2)
        nrows = (pos[-1] + 1 + _CHUNK - 1) // _CHUNK
        return (s3.reshape(-1, _CHUNK), d3.reshape(-1, _CHUNK),
                jnp.full((16,), nrows, jnp.int32))

    src_hi, dst_hi, n_hi = compact(dst >= _HALF, dst - _HALF)
    src_lo, dst_lo, n_lo = compact(dst < _HALF, dst)
    zeros_acc = jnp.zeros((_AROWS, _EMB), jnp.float32)

    l_msg = _mlp3(emb_l, L_msg)
    table1 = jnp.concatenate(
        [l_msg, jnp.zeros((_TROWS - _HALF, _EMB), jnp.float32)], axis=0)
    parts1 = _scatter(table1, src_hi, dst_hi, n_hi, zeros_acc)
    c_emb, c_msg = _clause_update(emb_c, parts1[0], C_update, C_msg)

    table2 = jnp.concatenate(
        [jnp.zeros((_HALF, _EMB), jnp.float32), c_msg,
         jnp.zeros((_TROWS - _N, _EMB), jnp.float32)], axis=0)
    parts2 = _scatter(table2, src_lo, dst_lo, n_lo, zeros_acc)
    l_emb = _literal_update(emb_l, parts2[0], l_msg, L_update)

    return jnp.concatenate([l_emb, c_emb], axis=0)


# speedup vs baseline: 7.4689x; 7.4689x over previous
import functools

import jax
import jax.numpy as jnp
from jax import lax
from jax.experimental import pallas as pl
from jax.experimental.pallas import tpu as pltpu
from jax.experimental.pallas import tpu_sc as plsc

_N = 10000
_HALF = 5000
_P = 2500
_EMB = 128
_E = 320000
_CHUNK = 120
_NCORE = 2
_NSUB = 16
_NW = _NCORE * _NSUB
_K0 = 168
_RING = 4
_EPROWS = _NSUB * _K0
_EP = _EPROWS * _CHUNK
_TROWS = 10240
_PAD_SRC = 10016
_AROWS = 5120


def _make_scatter_kernel():
    mesh = plsc.VectorSubcoreMesh(core_axis_name="c", subcore_axis_name="s")

    @functools.partial(
        pl.kernel,
        out_type=jax.ShapeDtypeStruct((1, _HALF, _EMB), jnp.float32),
        mesh=mesh,
        scratch_types=[
            pltpu.VMEM((_K0, _CHUNK), jnp.int32),
            pltpu.VMEM((16, _CHUNK), jnp.int32),
            pltpu.VMEM((_RING, _CHUNK, _EMB), jnp.float32),
            pltpu.VMEM_SHARED((_AROWS, _EMB), jnp.float32),
            pltpu.SemaphoreType.DMA, pltpu.SemaphoreType.DMA,
            pltpu.SemaphoreType.DMA, pltpu.SemaphoreType.DMA,
            pltpu.SemaphoreType.DMA, pltpu.SemaphoreType.DMA,
            pltpu.SemaphoreType.DMA, pltpu.SemaphoreType.DMA,
            pltpu.SemaphoreType.DMA,
        ],
    )
    def k(table_hbm, src_hbm, dst_hbm, zeros_hbm, out_hbm,
          src_v, dst_v, bufs, acc, *sems):
        gsem, ssem, dsem = sems[:_RING], sems[_RING:2 * _RING], sems[-1]
        core = lax.axis_index("c")
        sid = lax.axis_index("s")

        @pl.when(core == 0)
        def _():
            base = sid * _K0
            zrows = _AROWS // _NSUB
            pltpu.sync_copy(zeros_hbm.at[pl.ds(sid * zrows, zrows)],
                            acc.at[pl.ds(sid * zrows, zrows)])
            pltpu.sync_copy(src_hbm.at[pl.ds(base, _K0)], src_v)
            pltpu.sync_copy(dst_hbm.at[pl.ds(base, 8)], dst_v.at[pl.ds(0, 8)])
            pltpu.async_copy(dst_hbm.at[pl.ds(base + 8, 8)],
                             dst_v.at[pl.ds(8, 8)], dsem)
            for b in range(_RING):
                pltpu.async_copy(table_hbm.at[src_v.at[b]], bufs.at[b],
                                 gsem[b])

            @pl.loop(0, _K0, step=8)
            def _(ci):
                p = (ci // 8) % 2

                @pl.when(ci >= 8)
                def _():
                    pltpu.make_async_copy(dst_hbm.at[pl.ds(base, 8)],
                                          dst_v.at[pl.ds(0, 8)], dsem).wait()

                for half in range(2):
                    for b in range(_RING):
                        j = 4 * half + b
                        pltpu.make_async_copy(table_hbm.at[src_v.at[ci + j]],
                                              bufs.at[b], gsem[b]).wait()
                        pltpu.async_copy(bufs.at[b],
                                         acc.at[dst_v.at[p * 8 + j]],
                                         ssem[b], add=True)
                    for b in range(_RING):
                        j = 4 * half + b
                        k = ci + j
                        pltpu.make_async_copy(bufs.at[b],
                                              acc.at[dst_v.at[p * 8 + j]],
                                              ssem[b]).wait()

                        @pl.when(k + _RING < _K0)
                        def _():
                            pltpu.async_copy(table_hbm.at[src_v.at[k + _RING]],
                                             bufs.at[b], gsem[b])

                @pl.when(ci + 16 < _K0)
                def _():
                    pltpu.async_copy(dst_hbm.at[pl.ds(base + ci + 16, 8)],
                                     dst_v.at[pl.ds(p * 8, 8)], dsem)

        plsc.subcore_barrier()

        @pl.when((core == 0) & (sid < 5))
        def _():
            pltpu.sync_copy(acc.at[pl.ds(sid * 1000, 1000)],
                            out_hbm.at[0].at[pl.ds(sid * 1000, 1000)])

    return k


_scatter = _make_scatter_kernel()


def _full_spec():
    return pl.BlockSpec((_EMB, _EMB), lambda i: (0, 0))


def _bias_spec():
    return pl.BlockSpec((1, _EMB), lambda i: (0, 0))


def _row_spec(rows):
    return pl.BlockSpec((rows, _EMB), lambda i: (i, 0))


def _dot(a, b):
    return jnp.dot(a, b, preferred_element_type=jnp.float32)


def _mlp3(x, params):
    (w1, b1), (w2, b2), (w3, b3) = params
    rows = 1000
    n = x.shape[0]

    def body(x_ref, w1_ref, b1_ref, w2_ref, b2_ref, w3_ref, b3_ref, o_ref):
        h = jnp.maximum(_dot(x_ref[...], w1_ref[...]) + b1_ref[...], 0.0)
        h = jnp.maximum(_dot(h, w2_ref[...]) + b2_ref[...], 0.0)
        o_ref[...] = _dot(h, w3_ref[...]) + b3_ref[...]

    return pl.pallas_call(
        body,
        grid=(n // rows,),
        in_specs=[_row_spec(rows), _full_spec(), _bias_spec(), _full_spec(),
                  _bias_spec(), _full_spec(), _bias_spec()],
        out_specs=_row_spec(rows),
        out_shape=jax.ShapeDtypeStruct((n, _EMB), jnp.float32),
    )(x, w1, b1.reshape(1, _EMB), w2, b2.reshape(1, _EMB),
      w3, b3.reshape(1, _EMB))


def _clause_update(emb_c, p0, cu_params, cm_params):
    (wu1, bu1), (wu2, bu2), (wu3, bu3) = cu_params
    (wm1, bm1), (wm2, bm2), (wm3, bm3) = cm_params
    wu1a, wu1b = wu1[:_EMB], wu1[_EMB:]
    rows = 1000

    def body(e_ref, p0_ref, wu1a_ref, wu1b_ref, bu1_ref, wu2_ref,
             bu2_ref, wu3_ref, bu3_ref, wm1_ref, bm1_ref, wm2_ref, bm2_ref,
             wm3_ref, bm3_ref, ce_ref, cm_ref):
        h = _dot(e_ref[...], wu1a_ref[...]) + _dot(p0_ref[...], wu1b_ref[...])
        h = jnp.maximum(h + bu1_ref[...], 0.0)
        h = jnp.maximum(_dot(h, wu2_ref[...]) + bu2_ref[...], 0.0)
        ce = _dot(h, wu3_ref[...]) + bu3_ref[...]
        ce_ref[...] = ce
        m = jnp.maximum(_dot(ce, wm1_ref[...]) + bm1_ref[...], 0.0)
        m = jnp.maximum(_dot(m, wm2_ref[...]) + bm2_ref[...], 0.0)
        cm_ref[...] = _dot(m, wm3_ref[...]) + bm3_ref[...]

    return pl.pallas_call(
        body,
        grid=(_HALF // rows,),
        in_specs=[_row_spec(rows), _row_spec(rows),
                  _full_spec(), _full_spec(), _bias_spec(),
                  _full_spec(), _bias_spec(), _full_spec(), _bias_spec(),
                  _full_spec(), _bias_spec(), _full_spec(), _bias_spec(),
                  _full_spec(), _bias_spec()],
        out_specs=[_row_spec(rows), _row_spec(rows)],
        out_shape=[jax.ShapeDtypeStruct((_HALF, _EMB), jnp.float32),
                   jax.ShapeDtypeStruct((_HALF, _EMB), jnp.float32)],
    )(emb_c, p0, wu1a, wu1b, bu1.reshape(1, _EMB), wu2,
      bu2.reshape(1, _EMB), wu3, bu3.reshape(1, _EMB), wm1,
      bm1.reshape(1, _EMB), wm2, bm2.reshape(1, _EMB), wm3,
      bm3.reshape(1, _EMB))


def _literal_update(emb_l, q0, l_msg, lu_params):
    (wl1, bl1), (wl2, bl2), (wl3, bl3) = lu_params
    wl1a, wl1b, wl1c = wl1[:_EMB], wl1[_EMB:2 * _EMB], wl1[2 * _EMB:]

    def body(e_ref, q0_ref, f_ref, wl1a_ref, wl1b_ref, wl1c_ref,
             bl1_ref, wl2_ref, bl2_ref, wl3_ref, bl3_ref, o_ref):
        h = (_dot(e_ref[0], wl1a_ref[...]) + _dot(q0_ref[0], wl1b_ref[...])
             + _dot(f_ref[0], wl1c_ref[...]))
        h = jnp.maximum(h + bl1_ref[...], 0.0)
        h = jnp.maximum(_dot(h, wl2_ref[...]) + bl2_ref[...], 0.0)
        o_ref[0] = _dot(h, wl3_ref[...]) + bl3_ref[...]

    h3 = pl.BlockSpec((1, _P, _EMB), lambda j: (j, 0, 0))
    flip_spec = pl.BlockSpec((1, _P, _EMB), lambda j: ((j + 1) % 2, 0, 0))
    r3 = lambda a: a.reshape(2, _P, _EMB)
    out = pl.pallas_call(
        body,
        grid=(2,),
        in_specs=[h3, h3, flip_spec,
                  _full_spec(), _full_spec(), _full_spec(), _bias_spec(),
                  _full_spec(), _bias_spec(), _full_spec(), _bias_spec()],
        out_specs=h3,
        out_shape=jax.ShapeDtypeStruct((2, _P, _EMB), jnp.float32),
    )(r3(emb_l), r3(q0), r3(l_msg), wl1a, wl1b, wl1c,
      bl1.reshape(1, _EMB), wl2, bl2.reshape(1, _EMB), wl3,
      bl3.reshape(1, _EMB))
    return out.reshape(_HALF, _EMB)


def kernel(node_embedding, node_type, edge_index, L_msg, C_msg, L_update,
           C_update):
    del node_type
    emb_l = node_embedding[:_HALF]
    emb_c = node_embedding[_HALF:]
    src_pad = jnp.full((_EP - _E,), _PAD_SRC, dtype=jnp.int32)
    src_p = jnp.concatenate([edge_index[0], src_pad]).reshape(-1, _CHUNK)
    dst = edge_index[1]
    dump = _HALF + (jnp.arange(_EP, dtype=jnp.int32) % (_AROWS - _HALF))
    dst_hi = jnp.where(dst >= _HALF, dst - _HALF, dump[:_E])
    dst_lo = jnp.where(dst < _HALF, dst, dump[:_E])
    dst_hi = jnp.concatenate([dst_hi, dump[_E:]]).reshape(-1, _CHUNK)
    dst_lo = jnp.concatenate([dst_lo, dump[_E:]]).reshape(-1, _CHUNK)
    zeros_acc = jnp.zeros((_AROWS, _EMB), jnp.float32)

    l_msg = _mlp3(emb_l, L_msg)
    table1 = jnp.concatenate(
        [l_msg, jnp.zeros((_TROWS - _HALF, _EMB), jnp.float32)], axis=0)
    parts1 = _scatter(table1, src_p, dst_hi, zeros_acc)
    c_emb, c_msg = _clause_update(emb_c, parts1[0], C_update, C_msg)

    table2 = jnp.concatenate(
        [jnp.zeros((_HALF, _EMB), jnp.float32), c_msg,
         jnp.zeros((_TROWS - _N, _EMB), jnp.float32)], axis=0)
    parts2 = _scatter(table2, src_p, dst_lo, zeros_acc)
    l_emb = _literal_update(emb_l, parts2[0], l_msg, L_update)

    return jnp.concatenate([l_emb, c_emb], axis=0)
